# baseline (device time: 26786 ns/iter reference)
import jax
import jax.numpy as jnp
from jax import lax
from jax.experimental import pallas as pl
from jax.experimental.pallas import tpu as pltpu


def kernel(x, router, W1, W2):
    t_per, d = x.shape
    e_per, _, f = W1.shape

    def body(x_ref, r_ref, w1_ref, w2_ref, out_ref,
             x_send, x_peer, r_peer, w_send, w_recv, p_send, p_recv,
             w1b_ref, w2b_ref, send_sems, recv_sems):
        my_x = lax.axis_index("x")
        my_y = lax.axis_index("y")
        my_z = lax.axis_index("z")
        peer = (1 - my_x, my_y, my_z)

        barrier_sem = pltpu.get_barrier_semaphore()
        pl.semaphore_signal(barrier_sem, inc=1, device_id=peer,
                            device_id_type=pl.DeviceIdType.MESH)
        pl.semaphore_wait(barrier_sem, 1)

        x_send[...] = x_ref[...].astype(jnp.bfloat16)
        rdma_x = pltpu.make_async_remote_copy(
            src_ref=x_send, dst_ref=x_peer,
            send_sem=send_sems.at[0], recv_sem=recv_sems.at[0],
            device_id=peer, device_id_type=pl.DeviceIdType.MESH)
        rdma_x.start()
        rdma_r = pltpu.make_async_remote_copy(
            src_ref=r_ref, dst_ref=r_peer,
            send_sem=send_sems.at[1], recv_sem=recv_sems.at[1],
            device_id=peer, device_id_type=pl.DeviceIdType.MESH)
        rdma_r.start()

        w1b_ref[...] = w1_ref[...].astype(jnp.bfloat16)
        w2b_ref[...] = w2_ref[...].astype(jnp.bfloat16)

        xf = x_ref[...]
        g_mine = jnp.dot(xf, r_ref[...], precision=lax.Precision.HIGHEST)
        rdma_r.wait_recv()
        g_peer = jnp.dot(xf, r_peer[...], precision=lax.Precision.HIGHEST)

        g4 = jnp.concatenate([g_mine, g_peer], axis=1)
        m1 = jnp.max(g4, axis=1, keepdims=True)
        m2 = jnp.max(jnp.where(g4 >= m1, -jnp.inf, g4), axis=1, keepdims=True)
        w4 = (jnp.where(g4 >= m2, jnp.exp(g4 - m1), 0.0)
              / (1.0 + jnp.exp(m2 - m1)))

        w_send[...] = w4[:, e_per:2 * e_per]
        rdma_w = pltpu.make_async_remote_copy(
            src_ref=w_send, dst_ref=w_recv,
            send_sem=send_sems.at[2], recv_sem=recv_sems.at[2],
            device_id=peer, device_id_type=pl.DeviceIdType.MESH)
        rdma_w.start()

        def expert_sum(tokens_bf16, weights):
            acc = jnp.zeros((t_per, d), jnp.float32)
            for le in range(e_per):
                h = jnp.maximum(
                    jnp.dot(tokens_bf16, w1b_ref[le],
                            preferred_element_type=jnp.float32), 0.0)
                p = jnp.dot(h.astype(jnp.bfloat16), w2b_ref[le],
                            preferred_element_type=jnp.float32)
                acc = acc + p * weights[:, le:le + 1]
            return acc

        rdma_x.wait_recv()
        rdma_w.wait_recv()
        p_send[...] = expert_sum(x_peer[...], w_recv[...]).astype(jnp.bfloat16)
        rdma_p = pltpu.make_async_remote_copy(
            src_ref=p_send, dst_ref=p_recv,
            send_sem=send_sems.at[3], recv_sem=recv_sems.at[3],
            device_id=peer, device_id_type=pl.DeviceIdType.MESH)
        rdma_p.start()

        acc_my = expert_sum(x_send[...], w4[:, 0:e_per])

        rdma_p.wait_recv()
        out_ref[...] = acc_my + p_recv[...].astype(jnp.float32)

        rdma_x.wait_send()
        rdma_r.wait_send()
        rdma_w.wait_send()
        rdma_p.wait_send()

    return pl.pallas_call(
        body,
        out_shape=jax.ShapeDtypeStruct((t_per, d), jnp.float32),
        in_specs=[pl.BlockSpec(memory_space=pltpu.VMEM)] * 4,
        out_specs=pl.BlockSpec(memory_space=pltpu.VMEM),
        scratch_shapes=[
            pltpu.VMEM((t_per, d), jnp.bfloat16),
            pltpu.VMEM((t_per, d), jnp.bfloat16),
            pltpu.VMEM(router.shape, jnp.float32),
            pltpu.VMEM((t_per, e_per), jnp.float32),
            pltpu.VMEM((t_per, e_per), jnp.float32),
            pltpu.VMEM((t_per, d), jnp.bfloat16),
            pltpu.VMEM((t_per, d), jnp.bfloat16),
            pltpu.VMEM(W1.shape, jnp.bfloat16),
            pltpu.VMEM(W2.shape, jnp.bfloat16),
            pltpu.SemaphoreType.DMA((4,)),
            pltpu.SemaphoreType.DMA((4,)),
        ],
        compiler_params=pltpu.CompilerParams(collective_id=0),
    )(x, router, W1, W2)


# device time: 19839 ns/iter; 1.3502x vs baseline; 1.3502x over previous
import jax
import jax.numpy as jnp
from jax import lax
from jax.experimental import pallas as pl
from jax.experimental.pallas import tpu as pltpu


def kernel(x, router, W1, W2):
    t_per, d = x.shape
    e_per, _, f = W1.shape

    def body(x_ref, r_ref, w1_ref, w2_ref, out_ref,
             x_send, x_peer, r_send, r_peer, w_send, w_recv,
             p_send, p_recv, w1b_ref, w2b_ref, send_sems, recv_sems):
        my_x = lax.axis_index("x")
        my_y = lax.axis_index("y")
        my_z = lax.axis_index("z")
        peer = (1 - my_x, my_y, my_z)

        barrier_sem = pltpu.get_barrier_semaphore()
        pl.semaphore_signal(barrier_sem, inc=1, device_id=peer,
                            device_id_type=pl.DeviceIdType.MESH)
        pl.semaphore_wait(barrier_sem, 1)

        r_send[...] = r_ref[...].T
        rdma_r = pltpu.make_async_remote_copy(
            src_ref=r_send, dst_ref=r_peer,
            send_sem=send_sems.at[1], recv_sem=recv_sems.at[1],
            device_id=peer, device_id_type=pl.DeviceIdType.MESH)
        rdma_r.start()
        x_send[...] = x_ref[...].astype(jnp.bfloat16)
        rdma_x = pltpu.make_async_remote_copy(
            src_ref=x_send, dst_ref=x_peer,
            send_sem=send_sems.at[0], recv_sem=recv_sems.at[0],
            device_id=peer, device_id_type=pl.DeviceIdType.MESH)
        rdma_x.start()

        xf = x_ref[...]
        g_mine = jnp.dot(xf, r_ref[...], precision=lax.Precision.HIGHEST)
        rdma_r.wait_recv()
        g_peer = lax.dot_general(
            xf, r_peer[...], (((1,), (1,)), ((), ())),
            precision=lax.Precision.HIGHEST)

        g4 = jnp.concatenate([g_mine, g_peer], axis=1)
        m1 = jnp.max(g4, axis=1, keepdims=True)
        m2 = jnp.max(jnp.where(g4 >= m1, -jnp.inf, g4), axis=1, keepdims=True)
        e2 = jnp.exp(m2 - m1)
        w_top = 1.0 / (1.0 + e2)
        w4 = jnp.where(g4 >= m1, w_top,
                       jnp.where(g4 >= m2, 1.0 - w_top, 0.0))

        w_send[...] = w4[:, e_per:2 * e_per].T
        rdma_w = pltpu.make_async_remote_copy(
            src_ref=w_send, dst_ref=w_recv,
            send_sem=send_sems.at[2], recv_sem=recv_sems.at[2],
            device_id=peer, device_id_type=pl.DeviceIdType.MESH)
        rdma_w.start()

        w1b_ref[...] = w1_ref[...].astype(jnp.bfloat16)
        w2b_ref[...] = w2_ref[...].astype(jnp.bfloat16)

        def expert_sum(tokens_bf16, weights):
            acc = None
            for le in range(e_per):
                h = jnp.maximum(
                    jnp.dot(tokens_bf16, w1b_ref[le],
                            preferred_element_type=jnp.float32), 0.0)
                p = jnp.dot(h.astype(jnp.bfloat16), w2b_ref[le],
                            preferred_element_type=jnp.float32)
                p = p * weights[:, le:le + 1]
                acc = p if acc is None else acc + p
            return acc

        rdma_x.wait_recv()
        rdma_w.wait_recv()
        wp = w_recv[...].T
        half = t_per // 2
        rdma_p = []
        for i in range(2):
            sl = pl.ds(i * half, half)
            p_send[sl, :] = expert_sum(
                x_peer[sl, :], wp[i * half:(i + 1) * half, :]
            ).astype(jnp.bfloat16)
            r = pltpu.make_async_remote_copy(
                src_ref=p_send.at[sl], dst_ref=p_recv.at[sl],
                send_sem=send_sems.at[3 + i], recv_sem=recv_sems.at[3 + i],
                device_id=peer, device_id_type=pl.DeviceIdType.MESH)
            r.start()
            rdma_p.append(r)

        acc_my = expert_sum(x_send[...], w4[:, 0:e_per])

        for i, r in enumerate(rdma_p):
            r.wait_recv()
            sl = pl.ds(i * half, half)
            out_ref[sl, :] = (acc_my[i * half:(i + 1) * half, :]
                              + p_recv[sl, :].astype(jnp.float32))

        rdma_x.wait_send()
        rdma_r.wait_send()
        rdma_w.wait_send()
        for r in rdma_p:
            r.wait_send()

    return pl.pallas_call(
        body,
        out_shape=jax.ShapeDtypeStruct((t_per, d), jnp.float32),
        in_specs=[pl.BlockSpec(memory_space=pltpu.VMEM)] * 4,
        out_specs=pl.BlockSpec(memory_space=pltpu.VMEM),
        scratch_shapes=[
            pltpu.VMEM((t_per, d), jnp.bfloat16),
            pltpu.VMEM((t_per, d), jnp.bfloat16),
            pltpu.VMEM((e_per, d), jnp.float32),
            pltpu.VMEM((e_per, d), jnp.float32),
            pltpu.VMEM((e_per, t_per), jnp.float32),
            pltpu.VMEM((e_per, t_per), jnp.float32),
            pltpu.VMEM((t_per, d), jnp.bfloat16),
            pltpu.VMEM((t_per, d), jnp.bfloat16),
            pltpu.VMEM(W1.shape, jnp.bfloat16),
            pltpu.VMEM(W2.shape, jnp.bfloat16),
            pltpu.SemaphoreType.DMA((5,)),
            pltpu.SemaphoreType.DMA((5,)),
        ],
        compiler_params=pltpu.CompilerParams(collective_id=0),
    )(x, router, W1, W2)
